# baseline (speedup 1.0000x reference)
import jax
import jax.numpy as jnp
from jax import lax
from jax.experimental import pallas as pl
from jax.experimental.pallas import tpu as pltpu
from jax.experimental.pallas import tpu_sc as plsc

_NC, _NS = 2, 16
_NW = _NC * _NS


def kernel(input, embed_weight):
    L = input.shape[1]
    D = embed_weight.shape[1]
    per_w = L // _NW
    chunks = per_w // 128
    idx = input.reshape(_NW, chunks, 128).astype(jnp.int32)

    mesh = plsc.VectorSubcoreMesh(core_axis_name="c", subcore_axis_name="s")

    @pl.kernel(
        out_type=jax.ShapeDtypeStruct((L, D), embed_weight.dtype),
        mesh=mesh,
        compiler_params=pltpu.CompilerParams(
            needs_layout_passes=False, use_tc_tiling_on_sc=False
        ),
        scratch_types=[
            pltpu.VMEM((chunks, 128), jnp.int32),
            pltpu.VMEM((per_w, D), embed_weight.dtype),
            pltpu.SemaphoreType.DMA,
        ],
    )
    def _embed_kernel(table_hbm, idx_hbm, out_hbm, idx_v, rows_v, sem):
        wid = lax.axis_index("s") * _NC + lax.axis_index("c")
        base = wid * per_w
        pltpu.sync_copy(idx_hbm.at[wid], idx_v)
        copies = [
            pltpu.async_copy(
                table_hbm.at[idx_v.at[j]],
                rows_v.at[pl.ds(j * 128, 128)],
                sem,
            )
            for j in range(chunks)
        ]
        for c in copies:
            c.wait()
        pltpu.sync_copy(rows_v, out_hbm.at[pl.ds(base, per_w)])

    return _embed_kernel(embed_weight, idx)[None, :, :]


# speedup vs baseline: 2.9219x; 2.9219x over previous
import jax
import jax.numpy as jnp
from jax import lax
from jax.experimental import pallas as pl
from jax.experimental.pallas import tpu as pltpu
from jax.experimental.pallas import tpu_sc as plsc

_NC, _NS, _LANES = 2, 16, 16
_NW = _NC * _NS


def kernel(input, embed_weight):
    L = input.shape[1]
    D = embed_weight.shape[1]
    per_w = L // _NW
    groups = per_w // _LANES
    idx = input.reshape(L).astype(jnp.int32)

    mesh = plsc.VectorSubcoreMesh(core_axis_name="c", subcore_axis_name="s")

    @pl.kernel(
        out_type=jax.ShapeDtypeStruct((L * D,), embed_weight.dtype),
        mesh=mesh,
        compiler_params=pltpu.CompilerParams(needs_layout_passes=False),
        scratch_types=[
            pltpu.VMEM(embed_weight.shape, embed_weight.dtype),
            pltpu.VMEM((per_w,), jnp.int32),
            pltpu.VMEM((per_w * D,), embed_weight.dtype),
            pltpu.SemaphoreType.DMA,
            pltpu.SemaphoreType.DMA,
        ],
    )
    def _embed_kernel(table_hbm, idx_hbm, out_hbm, table_v, idx_v, out_v,
                      sem_t, sem_i):
        wid = lax.axis_index("s") * _NC + lax.axis_index("c")
        c_t = pltpu.async_copy(table_hbm, table_v, sem_t)
        c_i = pltpu.async_copy(idx_hbm.at[pl.ds(wid * per_w, per_w)], idx_v,
                               sem_i)
        lanes = lax.iota(jnp.int32, _LANES)
        s_pat = [lanes * D + d for d in range(D)]
        d_vec = [jnp.full((_LANES,), d, jnp.int32) for d in range(D)]
        c_i.wait()
        c_t.wait()

        @pl.loop(0, groups)
        def _(g):
            rows = idx_v[pl.ds(g * _LANES, _LANES)]
            base = g * (_LANES * D)
            for d in range(D):
                vals = plsc.load_gather(table_v, [rows, d_vec[d]])
                plsc.store_scatter(out_v, [s_pat[d] + base], vals)

        pltpu.sync_copy(out_v, out_hbm.at[pl.ds(wid * per_w * D, per_w * D)])

    return _embed_kernel(embed_weight, idx).reshape(1, L, D)
